# baseline (speedup 1.0000x reference)
import functools

import jax
import jax.numpy as jnp
from jax import lax
from jax.experimental import pallas as pl
from jax.experimental.pallas import tpu as pltpu
from jax.experimental.pallas import tpu_sc as plsc

D = 16
L = 16
NC = 2
NS = 16
NW = NC * NS
B = 16384
BPW = B // NW
GROUPS = BPW // L
GCHUNK = 128


def _sc_body(x_hbm, a_hbm, m_hbm, out_hbm, idx_v, x_v, rows_v, p_v, out_v, sem):
    wid = lax.axis_index("s") * NC + lax.axis_index("c")
    base = wid * BPW

    pltpu.sync_copy(a_hbm.at[pl.ds(base, BPW)], idx_v)
    copies = []
    for c in range(BPW // GCHUNK):
        copies.append(
            pltpu.async_copy(
                m_hbm.at[idx_v.at[pl.ds(c * GCHUNK, GCHUNK)]],
                rows_v.at[pl.ds(c * GCHUNK, GCHUNK)],
                sem,
            )
        )
    pltpu.sync_copy(x_hbm.at[pl.ds(base * D, BPW * D)], x_v)
    for cp in copies:
        cp.wait()

    lane = lax.iota(jnp.int32, L)

    def group(g, carry):
        g0 = pl.multiple_of(g * (L * D), L)
        for i in range(L):
            xrow = x_v[pl.ds(g0 + i * D, D)]
            grow = rows_v[g * L + i, :]
            p_v[pl.ds(g0 + i * D, D)] = xrow * grow
        flat0 = g0 + lane * D
        acc = plsc.load_gather(p_v, [flat0])
        for d in range(1, D):
            acc = acc + plsc.load_gather(p_v, [flat0 + d])
        out_v[pl.ds(pl.multiple_of(g * L, L), L)] = 1.0 / (1.0 + jnp.exp(-acc))
        return carry

    lax.fori_loop(0, GROUPS, group, 0)
    pltpu.sync_copy(out_v, out_hbm.at[pl.ds(base, BPW)])


_sc_call = functools.partial(
    pl.kernel,
    out_type=jax.ShapeDtypeStruct((B,), jnp.float32),
    mesh=plsc.VectorSubcoreMesh(core_axis_name="c", subcore_axis_name="s"),
    scratch_types=[
        pltpu.VMEM((BPW,), jnp.int32),
        pltpu.VMEM((BPW * D,), jnp.float32),
        pltpu.VMEM((BPW, D), jnp.float32),
        pltpu.VMEM((BPW * D,), jnp.float32),
        pltpu.VMEM((BPW,), jnp.float32),
        pltpu.SemaphoreType.DMA,
    ],
    compiler_params=pltpu.CompilerParams(
        needs_layout_passes=False, use_tc_tiling_on_sc=False
    ),
)(_sc_body)


@jax.jit
def kernel(X, A, m):
    return _sc_call(X.reshape(-1), A.astype(jnp.int32), m)
